# Initial kernel scaffold; baseline (speedup 1.0000x reference)
import jax
import jax.numpy as jnp
from jax.experimental import pallas as pl


def kernel(features, mask, node_embeds, hyper_embeds):
    raise NotImplementedError("write your pallas kernel here")


# speedup vs baseline: 1.1621x; 1.1621x over previous
import jax
import jax.numpy as jnp
from jax.experimental import pallas as pl
from jax.experimental.pallas import tpu as pltpu

_ALPHA = 1.0
_TOPK = 3
_BLOCK = 1024
_NEG = -1e9


def _hyper_kernel(mask_ref, ne_ref, hyt_ref, val_ref, idx_ref):
    i = pl.program_id(0)
    ne = ne_ref[...]
    hyt = hyt_ref[...]
    sim = jax.lax.dot_general(
        ne, hyt, (((1,), (0,)), ((), ())),
        preferred_element_type=jnp.float32)
    sim = jnp.maximum(_ALPHA * sim, 0.0)
    avg = jnp.mean(mask_ref[...], axis=0)
    sim = jnp.where(avg[:, None] < 0.5, _NEG, sim)
    m = jnp.max(sim, axis=1, keepdims=True)
    e = jnp.exp(sim - m)
    soft = e / jnp.sum(e, axis=1, keepdims=True)

    h = soft.shape[1]
    col = jax.lax.broadcasted_iota(jnp.int32, soft.shape, 1)
    v = soft
    for k in range(_TOPK):
        mk = jnp.max(v, axis=1)
        ik = jnp.min(jnp.where(v == mk[:, None], col, h), axis=1)
        val_ref[k, :] = mk
        idx_ref[k, :] = ik
        v = jnp.where(col == ik[:, None], -1.0, v)

    node_ids = jax.lax.iota(jnp.int32, ne.shape[0]) + i * ne.shape[0]
    for k in range(_TOPK):
        idx_ref[_TOPK + k, :] = node_ids


def kernel(features, mask, node_embeds, hyper_embeds):
    seq_len = min(features.shape[1], node_embeds.shape[0])
    ne = node_embeds[:seq_len]
    dim = ne.shape[1]
    hnum = hyper_embeds.shape[0]
    nblk = seq_len // _BLOCK

    vals, idxs = pl.pallas_call(
        _hyper_kernel,
        grid=(nblk,),
        in_specs=[
            pl.BlockSpec((mask.shape[0], _BLOCK), lambda i: (0, i)),
            pl.BlockSpec((_BLOCK, dim), lambda i: (i, 0)),
            pl.BlockSpec((dim, hnum), lambda i: (0, 0)),
        ],
        out_specs=[
            pl.BlockSpec((8, _BLOCK), lambda i: (0, i)),
            pl.BlockSpec((8, _BLOCK), lambda i: (0, i)),
        ],
        out_shape=[
            jax.ShapeDtypeStruct((8, seq_len), jnp.float32),
            jax.ShapeDtypeStruct((8, seq_len), jnp.int32),
        ],
    )(mask, ne, hyper_embeds.T)

    edge_weight = vals[:_TOPK].T.reshape(-1)
    col = idxs[:_TOPK].T.reshape(-1)
    row = idxs[_TOPK:2 * _TOPK].T.reshape(-1)
    edge_index = jnp.stack([row, col], axis=0)
    return (edge_index, edge_weight)


# speedup vs baseline: 2.0133x; 1.7324x over previous
import jax
import jax.numpy as jnp
from jax.experimental import pallas as pl
from jax.experimental.pallas import tpu as pltpu

_ALPHA = 1.0
_TOPK = 3
_BLOCK = 1024
_NEG = -1e9


def _hyper_kernel(mask_ref, ne_ref, hy_ref, val_ref, idx_ref):
    i = pl.program_id(0)
    ne = ne_ref[...]
    hy = hy_ref[...]
    simt = jax.lax.dot_general(
        hy, ne, (((1,), (1,)), ((), ())),
        preferred_element_type=jnp.float32)
    simt = jnp.maximum(_ALPHA * simt, 0.0)
    avg = jnp.mean(mask_ref[...], axis=0)
    simt = jnp.where(avg[None, :] < 0.5, _NEG, simt)
    m = jnp.max(simt, axis=0, keepdims=True)
    e = jnp.exp(simt - m)
    soft = e / jnp.sum(e, axis=0, keepdims=True)

    h = soft.shape[0]
    row = jax.lax.broadcasted_iota(jnp.int32, soft.shape, 0)
    v = soft
    for k in range(_TOPK):
        mk = jnp.max(v, axis=0)
        ik = jnp.min(jnp.where(v == mk[None, :], row, h), axis=0)
        val_ref[k, :] = mk
        idx_ref[k, :] = ik
        v = jnp.where(row == ik[None, :], -1.0, v)

    node_ids = jax.lax.iota(jnp.int32, ne.shape[0]) + i * ne.shape[0]
    for k in range(_TOPK):
        idx_ref[_TOPK + k, :] = node_ids


def kernel(features, mask, node_embeds, hyper_embeds):
    seq_len = min(features.shape[1], node_embeds.shape[0])
    ne = node_embeds[:seq_len]
    dim = ne.shape[1]
    hnum = hyper_embeds.shape[0]
    nblk = seq_len // _BLOCK

    vals, idxs = pl.pallas_call(
        _hyper_kernel,
        grid=(nblk,),
        in_specs=[
            pl.BlockSpec((mask.shape[0], _BLOCK), lambda i: (0, i)),
            pl.BlockSpec((_BLOCK, dim), lambda i: (i, 0)),
            pl.BlockSpec((hnum, dim), lambda i: (0, 0)),
        ],
        out_specs=[
            pl.BlockSpec((8, _BLOCK), lambda i: (0, i)),
            pl.BlockSpec((8, _BLOCK), lambda i: (0, i)),
        ],
        out_shape=[
            jax.ShapeDtypeStruct((8, seq_len), jnp.float32),
            jax.ShapeDtypeStruct((8, seq_len), jnp.int32),
        ],
    )(mask, ne, hyper_embeds)

    edge_weight = vals[:_TOPK].T.reshape(-1)
    col = idxs[:_TOPK].T.reshape(-1)
    row = idxs[_TOPK:2 * _TOPK].T.reshape(-1)
    edge_index = jnp.stack([row, col], axis=0)
    return (edge_index, edge_weight)
